# baseline (speedup 1.0000x reference)
import jax
import jax.numpy as jnp
import numpy as np
from jax import lax
from jax.experimental import pallas as pl
from jax.experimental.pallas import tpu as pltpu, tpu_sc as plsc

B = 4096
L = 50
D = 64
PACK = 4
LANES = 16
NC, NS = 2, 16
NW = NC * NS
GROUPS = B // (NW * LANES)
LPAD = 64
JU = 16
UNITS = LPAD // JU
ROWS_BUF = LANES * JU


def _sc_body(ids_hbm, idm_hbm, w_hbm, table_hbm, out_hbm, idx0, idx1,
             idm0, idm1, w0, w1, rows_a, rows_b, out_v, sem_a, sem_b, sem_s):
    cid = lax.axis_index("c")
    sid = lax.axis_index("s")
    wid = sid * NC + cid

    idx_bufs = (idx0, idx1)
    idm_bufs = (idm0, idm1)
    w_bufs = (w0, w1)
    rows_sems = ((rows_a, sem_a), (rows_b, sem_b))
    iota = jax.lax.iota(jnp.int32, LANES)

    def stage(src_g, parity, sync):
        b0 = (wid * GROUPS + src_g) * LANES
        if sync:
            pltpu.sync_copy(ids_hbm.at[pl.ds(b0, LANES)], idx_bufs[parity])
            pltpu.sync_copy(idm_hbm.at[pl.ds(b0, LANES)], idm_bufs[parity])
            pltpu.sync_copy(w_hbm.at[pl.ds(b0, LANES)], w_bufs[parity])
            return ()
        return (
            pltpu.async_copy(ids_hbm.at[pl.ds(b0, LANES)], idx_bufs[parity],
                             sem_s),
            pltpu.async_copy(idm_hbm.at[pl.ds(b0, LANES)], idm_bufs[parity],
                             sem_s),
            pltpu.async_copy(w_hbm.at[pl.ds(b0, LANES)], w_bufs[parity],
                             sem_s),
        )

    def fire(gp, q, u):
        idx_v = idx_bufs[gp]
        rows_buf, sem = rows_sems[u]
        for b in range(LANES):
            pltpu.async_copy(
                table_hbm.at[idx_v.at[b, pl.ds(q * JU, JU)]],
                rows_buf.at[pl.ds(b * JU, JU), :],
                sem,
            )

    def drain(u):
        rows_buf, sem = rows_sems[u]
        pltpu.make_async_copy(
            table_hbm.at[pl.ds(0, ROWS_BUF)], rows_buf, sem).wait()

    def accumulate(gp, q, u):
        idm_v = idm_bufs[gp]
        w_v = w_bufs[gp]
        rows_buf = rows_sems[u][0]

        def b_body(b, carry):
            idc = idm_v[b, pl.ds(q * JU, JU)]
            wc = w_v[b, pl.ds(q * JU, JU)]
            accs = [jnp.zeros((LANES,), jnp.float32) for _ in range(4)]
            for j in range(JU):
                lane = jnp.full((LANES,), j, jnp.int32)
                dn = lax.GatherDimensionNumbers(
                    offset_dims=(), collapsed_slice_dims=(0,),
                    start_index_map=(0,))
                wb = lax.gather(wc, lane[:, None], dimension_numbers=dn,
                                slice_sizes=(1,),
                                mode=lax.GatherScatterMode.PROMISE_IN_BOUNDS)
                idj = lax.gather(idc, lane[:, None], dimension_numbers=dn,
                                 slice_sizes=(1,),
                                 mode=lax.GatherScatterMode.PROMISE_IN_BOUNDS)
                col0 = (idj & 3) * 32 + iota
                rsp = jnp.full((LANES,), 0, jnp.int32) + (b * JU + j)
                v0 = plsc.load_gather(rows_buf, [rsp, col0])
                v1 = plsc.load_gather(rows_buf, [rsp, col0 + 16])
                e0, o0 = plsc.unpack(plsc.bitcast(v0, jnp.bfloat16),
                                     format=plsc.PackFormat.INTERLEAVED,
                                     preferred_element_type=jnp.float32)
                e1, o1 = plsc.unpack(plsc.bitcast(v1, jnp.bfloat16),
                                     format=plsc.PackFormat.INTERLEAVED,
                                     preferred_element_type=jnp.float32)
                accs[0] = accs[0] + wb * e0
                accs[1] = accs[1] + wb * o0
                accs[2] = accs[2] + wb * e1
                accs[3] = accs[3] + wb * o1
            for k in range(4):
                if q == 0:
                    out_v[b, pl.ds(16 * k, 16)] = accs[k]
                else:
                    plsc.addupdate(out_v.at[b, pl.ds(16 * k, 16)], accs[k])
            return carry

        lax.fori_loop(0, LANES, b_body, 0)

    def do_group(g, gp):
        for q in range(UNITS):
            u = q % 2
            drain(u)
            if q + 1 < UNITS:
                fire(gp, q + 1, (q + 1) % 2)
            elif q + 1 == UNITS:
                fire(1 - gp, 0, 0)
            accumulate(gp, q, u)
        b0 = (wid * GROUPS + g) * LANES
        pltpu.sync_copy(out_v, out_hbm.at[pl.ds(b0, LANES)])

    stage(0, 0, sync=True)
    fire(0, 0, 0)

    def gg_body(gg, carry):
        g_even = 2 * gg
        g_odd = 2 * gg + 1
        s1 = stage(g_odd, 1, sync=False)
        for c in s1:
            c.wait()
        do_group(g_even, 0)
        s2 = stage(jnp.minimum(g_odd + 1, GROUPS - 1), 0, sync=False)
        for c in s2:
            c.wait()
        do_group(g_odd, 1)
        return carry

    lax.fori_loop(0, GROUPS // 2, gg_body, 0)
    drain(0)


_sc_kernel = pl.kernel(
    _sc_body,
    out_type=jax.ShapeDtypeStruct((B, D), jnp.float32),
    mesh=plsc.VectorSubcoreMesh(core_axis_name="c", subcore_axis_name="s"),
    scratch_types=[
        pltpu.VMEM((LANES, LPAD), jnp.int32),
        pltpu.VMEM((LANES, LPAD), jnp.int32),
        pltpu.VMEM((LANES, LPAD), jnp.int32),
        pltpu.VMEM((LANES, LPAD), jnp.int32),
        pltpu.VMEM((LANES, LPAD), jnp.float32),
        pltpu.VMEM((LANES, LPAD), jnp.float32),
        pltpu.VMEM((ROWS_BUF, 128), jnp.int32),
        pltpu.VMEM((ROWS_BUF, 128), jnp.int32),
        pltpu.VMEM((LANES, D), jnp.float32),
        pltpu.SemaphoreType.DMA,
        pltpu.SemaphoreType.DMA,
        pltpu.SemaphoreType.DMA,
    ],
    compiler_params=pltpu.CompilerParams(
        use_tc_tiling_on_sc=True, needs_layout_passes=False
    ),
)


_INV_COLS = np.array(
    [(0 if d < 32 else 32) + (0 if d % 2 == 0 else 16) + (d % 32) // 2
     for d in range(D)],
    dtype=np.int32,
)


def kernel(sp_ids, sp_weights, embeddings):
    emb_bf = embeddings.astype(jnp.bfloat16)
    packed = lax.bitcast_convert_type(
        emb_bf.reshape(1000000 // PACK, 128, 2), jnp.int32)
    ids_pad = jnp.pad(sp_ids, ((0, 0), (0, LPAD - L)))
    ids_div = ids_pad // PACK
    w_pad = jnp.pad(sp_weights, ((0, 0), (0, LPAD - L)))
    out_perm = _sc_kernel(ids_div, ids_pad, w_pad, packed)
    return out_perm[:, _INV_COLS]


# speedup vs baseline: 39.9983x; 39.9983x over previous
import jax
import jax.numpy as jnp
from jax import lax
from jax.experimental import pallas as pl
from jax.experimental.pallas import tpu as pltpu, tpu_sc as plsc

B = 4096
L = 50
D = 64
DPAD = 128
LANES = 16
NC, NS = 2, 16
NW = NC * NS
GROUPS = B // (NW * LANES)
IDS_PER_GROUP = LANES * L


LH = L // 2
WPAD = 64


def _sc_body(ids_hbm, w_hbm, table_hbm, out_hbm, idx0, idx1, w0, w1,
             rows_a, rows_b, out0, out1, sem_a, sem_b, sem_s, sem_o):
    cid = lax.axis_index("c")
    sid = lax.axis_index("s")
    wid = sid * NC + cid

    idx_bufs = (idx0, idx1)
    w_bufs = (w0, w1)
    out_bufs = (out0, out1)

    def stage(g, sync):
        b0 = (wid * GROUPS + g) * LANES
        p = g % 2
        if sync:
            pltpu.sync_copy(ids_hbm.at[pl.ds(b0, LANES)], idx_bufs[p])
            pltpu.sync_copy(w_hbm.at[pl.ds(b0, LANES)], w_bufs[p])
            return ()
        return (
            pltpu.async_copy(ids_hbm.at[pl.ds(b0, LANES)], idx_bufs[p], sem_s),
            pltpu.async_copy(w_hbm.at[pl.ds(b0, LANES)], w_bufs[p], sem_s),
        )

    def fire(g, half, rows_buf, sem):
        idx_v = idx_bufs[g % 2]
        return [
            pltpu.async_copy(
                table_hbm.at[idx_v.at[b, pl.ds(half * LH, LH)]],
                rows_buf.at[pl.ds(b * LH, LH), :],
                sem,
            )
            for b in range(LANES)
        ]

    def accumulate(g, half, rows_buf):
        w_v = w_bufs[g % 2]
        out_v = out_bufs[g % 2]

        def b_body(b, carry):
            chunks = {
                c: w_v[b, pl.ds(16 * c, 16)]
                for c in range((half * LH) // 16, (half * LH + LH - 1) // 16 + 1)
            }
            accs = [jnp.zeros((LANES,), jnp.float32) for _ in range(D // LANES)]
            for j_local in range(LH):
                j_abs = half * LH + j_local
                lane = jnp.full((LANES,), j_abs % 16, jnp.int32)
                wb = lax.gather(
                    chunks[j_abs // 16], lane[:, None],
                    dimension_numbers=lax.GatherDimensionNumbers(
                        offset_dims=(), collapsed_slice_dims=(0,),
                        start_index_map=(0,)),
                    slice_sizes=(1,),
                    mode=lax.GatherScatterMode.PROMISE_IN_BOUNDS)
                row = b * LH + j_local
                for db in range(D // LANES):
                    vals = rows_buf[row, pl.ds(db * 16, 16)]
                    accs[db] = accs[db] + wb * vals
            for db in range(D // LANES):
                if half == 0:
                    out_v[b, pl.ds(db * 16, 16)] = accs[db]
                else:
                    plsc.addupdate(out_v.at[b, pl.ds(db * 16, 16)], accs[db])
            return carry

        lax.fori_loop(0, LANES, b_body, 0)

    stage(0, sync=True)
    copies_a = fire(0, 0, rows_a, sem_a)
    copies_b = fire(0, 1, rows_b, sem_b)
    out_copies = [None, None]
    for g in range(GROUPS):
        staging = stage(g + 1, sync=False) if g + 1 < GROUPS else ()
        for c in copies_a:
            c.wait()
        if out_copies[g % 2] is not None:
            out_copies[g % 2].wait()
            out_copies[g % 2] = None
        accumulate(g, 0, rows_a)
        for c in copies_b:
            c.wait()
        if g + 1 < GROUPS:
            for c in staging:
                c.wait()
            copies_a = fire(g + 1, 0, rows_a, sem_a)
        accumulate(g, 1, rows_b)
        if g + 1 < GROUPS:
            copies_b = fire(g + 1, 1, rows_b, sem_b)
        b0 = (wid * GROUPS + g) * LANES
        out_copies[g % 2] = pltpu.async_copy(
            out_bufs[g % 2], out_hbm.at[pl.ds(b0, LANES)], sem_o)
    for c in out_copies:
        if c is not None:
            c.wait()


_sc_kernel = pl.kernel(
    _sc_body,
    out_type=jax.ShapeDtypeStruct((B, D), jnp.float32),
    mesh=plsc.VectorSubcoreMesh(core_axis_name="c", subcore_axis_name="s"),
    scratch_types=[
        pltpu.VMEM((LANES, L), jnp.int32),
        pltpu.VMEM((LANES, L), jnp.int32),
        pltpu.VMEM((LANES, WPAD), jnp.float32),
        pltpu.VMEM((LANES, WPAD), jnp.float32),
        pltpu.VMEM((IDS_PER_GROUP // 2, DPAD), jnp.float32),
        pltpu.VMEM((IDS_PER_GROUP // 2, DPAD), jnp.float32),
        pltpu.VMEM((LANES, D), jnp.float32),
        pltpu.VMEM((LANES, D), jnp.float32),
        pltpu.SemaphoreType.DMA,
        pltpu.SemaphoreType.DMA,
        pltpu.SemaphoreType.DMA,
        pltpu.SemaphoreType.DMA,
    ],
    compiler_params=pltpu.CompilerParams(
        use_tc_tiling_on_sc=True, needs_layout_passes=False
    ),
)


def kernel(sp_ids, sp_weights, embeddings):
    emb_pad = jnp.pad(embeddings, ((0, 0), (0, DPAD - D)))
    w_pad = jnp.pad(sp_weights, ((0, 0), (0, WPAD - L)))
    return _sc_kernel(sp_ids, w_pad, emb_pad)
